# baseline (speedup 1.0000x reference)
import functools

import jax
import jax.numpy as jnp
from jax import lax
from jax.experimental import pallas as pl
from jax.experimental.pallas import tpu as pltpu
from jax.experimental.pallas import tpu_sc as plsc

N_NODES = 10000
NPAD = 10240
RB = 512
NBLK = NPAD // RB
E_TOTAL = 640000
NSUB = 16
NCORE = 2
KCH = 112
G = 64
EPS = 1e-16
LRELU_SLOPE = 0.2

NBUF = 5
LOOK = 3
NCHUNK = 360
EPAD = NSUB * NCHUNK * KCH
PAD_NODE = N_NODES + 100
ITERS = NCHUNK // NBUF


def _nq(f_out):
    if f_out >= 256:
        return 8
    return 4 if f_out >= 64 else 2


def _lrelu(t):
    return jnp.where(t > 0, t, LRELU_SLOPE * t)


def _attn_tail(xp, a_s, a_d, i, xp_refs, es_ref, ed_ref, m_ref,
               ed_sc, gmax_sm):
    nq = len(xp_refs)
    fq = xp.shape[1] // nq

    @pl.when(i < NBLK)
    def _():
        es = jnp.sum(xp * a_s, axis=1, keepdims=True)
        ed = jnp.sum(xp * a_d, axis=1, keepdims=True)
        for q, r in enumerate(xp_refs):
            r[...] = xp[:, q * fq:(q + 1) * fq]
        es_ref[...] = es
        ed_ref[...] = ed
        ed_sc[pl.ds(i * RB, RB), :] = ed
        cur = jnp.max(es)
        prev = jnp.where(i == 0, -jnp.inf, gmax_sm[0])
        gmax_sm[0] = jnp.maximum(prev, cur)

    @pl.when(i == NBLK)
    def _():
        m_ref[...] = _lrelu(gmax_sm[0] + ed_sc[...])


def _prep_body(*refs, nq_out):
    (x_ref, w1, b1, w2, b2, w3, b3, wg, a_s, a_d) = refs[:10]
    xp_refs = refs[10:10 + nq_out]
    es_ref, ed_ref, m_ref, ed_sc, gmax_sm = refs[10 + nq_out:]
    i = pl.program_id(0)
    h = jax.nn.relu(jnp.dot(x_ref[...], w1[...],
                            preferred_element_type=jnp.float32) + b1[...])
    h = jax.nn.relu(jnp.dot(h, w2[...], preferred_element_type=jnp.float32)
                    + b2[...])
    h = jax.nn.relu(jnp.dot(h, w3[...], preferred_element_type=jnp.float32)
                    + b3[...])
    xp = jnp.dot(h, wg[...], preferred_element_type=jnp.float32)
    _attn_tail(xp, a_s[...], a_d[...], i, xp_refs, es_ref, ed_ref, m_ref,
               ed_sc, gmax_sm)


def _mid_body(*refs, nq_in, nq_out):
    n_refs = refs[:nq_in]
    den_ref = refs[nq_in]
    b_refs = refs[nq_in + 1:2 * nq_in + 1]
    w_refs = refs[2 * nq_in + 1:3 * nq_in + 1]
    a_s, a_d = refs[3 * nq_in + 1:3 * nq_in + 3]
    xp_refs = refs[3 * nq_in + 3:3 * nq_in + 3 + nq_out]
    es_ref, ed_ref, m_ref, ed_sc, gmax_sm = refs[3 * nq_in + 3 + nq_out:]
    i = pl.program_id(0)
    d = den_ref[...] + EPS
    xp = None
    for q in range(nq_in):
        h = jax.nn.relu(n_refs[q][...] / d + b_refs[q][...])
        part = jnp.dot(h, w_refs[q][...], preferred_element_type=jnp.float32)
        xp = part if xp is None else xp + part
    _attn_tail(xp, a_s[...], a_d[...], i, xp_refs, es_ref, ed_ref, m_ref,
               ed_sc, gmax_sm)


def _final_body(*refs, nq_in):
    n_refs = refs[:nq_in]
    den_ref = refs[nq_in]
    b_refs = refs[nq_in + 1:2 * nq_in + 1]
    batch_ref, wl, bl, out_ref, sums_sc, cnt_sc = refs[2 * nq_in + 1:]
    i = pl.program_id(0)
    fq = n_refs[0].shape[1]

    @pl.when(i == 0)
    def _():
        sums_sc[...] = jnp.zeros_like(sums_sc)
        cnt_sc[...] = jnp.zeros_like(cnt_sc)

    @pl.when(i < NBLK)
    def _():
        d = den_ref[...] + EPS
        iota = lax.broadcasted_iota(jnp.int32, (RB, G), 1)
        oh = (batch_ref[...] == iota).astype(jnp.float32)
        dn = (((0,), (0,)), ((), ()))
        for q in range(nq_in):
            h = jax.nn.relu(n_refs[q][...] / d + b_refs[q][...])
            sums_sc[:, q * fq:(q + 1) * fq] += lax.dot_general(
                oh, h, dn, preferred_element_type=jnp.float32)
        ones = jnp.ones((RB, 1), jnp.float32)
        cnt_sc[...] += lax.dot_general(oh, ones, dn,
                                       preferred_element_type=jnp.float32)

    @pl.when(i == NBLK)
    def _():
        pooled = sums_sc[...] / jnp.maximum(cnt_sc[...], 1.0)
        z = jnp.dot(pooled, wl[...], preferred_element_type=jnp.float32) \
            + bl[...]
        out_ref[...] = jax.nn.sigmoid(z)


def _clamp_map(i):
    return (jnp.minimum(i, NBLK - 1), 0)


def _full(shape):
    return pl.BlockSpec(shape, lambda i: tuple(0 for _ in shape))


def _node_specs(f_out):
    nq = _nq(f_out)
    fq = f_out // nq
    out_shape = [jax.ShapeDtypeStruct((NPAD, fq), jnp.float32)
                 for _ in range(nq)] + [
        jax.ShapeDtypeStruct((NPAD, 1), jnp.float32),
        jax.ShapeDtypeStruct((NPAD, 1), jnp.float32),
        jax.ShapeDtypeStruct((NPAD, 1), jnp.float32),
    ]
    out_specs = [pl.BlockSpec((RB, fq), _clamp_map) for _ in range(nq)] + [
        pl.BlockSpec((RB, 1), _clamp_map),
        pl.BlockSpec((RB, 1), _clamp_map),
        pl.BlockSpec((NPAD, 1), lambda i: (0, 0)),
    ]
    scratch = [pltpu.VMEM((NPAD, 1), jnp.float32),
               pltpu.SMEM((1,), jnp.float32)]
    return out_shape, out_specs, scratch


def _tc_prep(x, w1, b1, w2, b2, w3, b3, wg, a_s, a_d):
    f_out = wg.shape[1]
    nq = _nq(f_out)
    out_shape, out_specs, scratch = _node_specs(f_out)
    args = (x, w1, b1, w2, b2, w3, b3, wg, a_s, a_d)
    in_specs = [pl.BlockSpec((RB, 1), _clamp_map)] + [
        _full(a.shape) for a in args[1:]]
    return pl.pallas_call(
        functools.partial(_prep_body, nq_out=nq),
        grid=(NBLK + 1,),
        in_specs=in_specs,
        out_specs=out_specs,
        out_shape=out_shape,
        scratch_shapes=scratch,
    )(*args)


def _tc_mid(n_parts, den, b_parts, w_parts, a_s, a_d):
    nq_in = len(n_parts)
    f_out = w_parts[0].shape[1]
    nq_out = _nq(f_out)
    fq_in = n_parts[0].shape[1]
    out_shape, out_specs, scratch = _node_specs(f_out)
    in_specs = ([pl.BlockSpec((RB, fq_in), _clamp_map)] * nq_in
                + [pl.BlockSpec((RB, 1), _clamp_map)]
                + [_full(a.shape) for a in (*b_parts, *w_parts, a_s, a_d)])
    return pl.pallas_call(
        functools.partial(_mid_body, nq_in=nq_in, nq_out=nq_out),
        grid=(NBLK + 1,),
        in_specs=in_specs,
        out_specs=out_specs,
        out_shape=out_shape,
        scratch_shapes=scratch,
    )(*n_parts, den, *b_parts, *w_parts, a_s, a_d)


def _tc_final(n_parts, den, b_parts, batch, wl, bl):
    nq_in = len(n_parts)
    fq_in = n_parts[0].shape[1]
    in_specs = ([pl.BlockSpec((RB, fq_in), _clamp_map)] * nq_in
                + [pl.BlockSpec((RB, 1), _clamp_map)]
                + [_full(a.shape) for a in b_parts]
                + [pl.BlockSpec((RB, 1), _clamp_map),
                   _full(wl.shape), _full(bl.shape)])
    return pl.pallas_call(
        functools.partial(_final_body, nq_in=nq_in),
        grid=(NBLK + 1,),
        in_specs=in_specs,
        out_specs=pl.BlockSpec((G, 1), lambda i: (0, 0)),
        out_shape=jax.ShapeDtypeStruct((G, 1), jnp.float32),
        scratch_shapes=[pltpu.VMEM((G, nq_in * fq_in), jnp.float32),
                        pltpu.VMEM((G, 1), jnp.float32)],
    )(*n_parts, den, *b_parts, batch, wl, bl)


def _sc_edge_body(*refs, f_scq, n_pass):
    nq = 2 * n_pass
    (src_hbm, dst_hbm, es_hbm, ed_hbm, m_hbm) = refs[:5]
    xps = refs[5:5 + nq]
    numer_hbm, den_hbm = refs[5 + nq:7 + nq]
    (src2, dst2, esb, edb, mb, exb, rowsb,
     sg0, sg1, sg2, sg3, sg4, ss0, ss1, ss2, ss3, ss4,
     acc_sp, den_sp) = refs[7 + nq:]
    c = lax.axis_index("c")
    s = lax.axis_index("s")
    sg = (sg0, sg1, sg2, sg3, sg4)
    ss = (ss0, ss1, ss2, ss3, ss4)
    rows_per_tile = NPAD // NSUB
    base_row = s * rows_per_tile
    fb = f_scq // 16

    pltpu.sync_copy(src_hbm.at[s], src2)
    pltpu.sync_copy(dst_hbm.at[s], dst2)

    def _zero_acc(zero_den):
        def _zero_rows(j, _):
            for f in range(fb):
                rowsb[0, j, pl.ds(f * 16, 16)] = jnp.zeros((16,), jnp.float32)
            return 0

        lax.fori_loop(0, KCH, _zero_rows, 0)
        offs = [z * KCH for z in range(rows_per_tile // KCH)]
        if (rows_per_tile // KCH) * KCH < rows_per_tile:
            offs.append(rows_per_tile - KCH)
        for o in offs:
            pltpu.sync_copy(rowsb.at[0],
                            acc_sp.at[pl.ds(base_row + o, KCH)])
        if zero_den:
            for z in range(KCH // 16):
                exb[0, pl.ds(z * 16, 16)] = jnp.zeros((16,), jnp.float32)
            for o in offs:
                pltpu.sync_copy(exb.at[0],
                                den_sp.at[pl.ds(base_row + o, KCH)])

    def _issue_gathers(ci, b, tab0, tab1):
        si = src2.at[ci]
        di = dst2.at[ci]
        pltpu.async_copy(es_hbm.at[si], esb.at[b], sg[b])
        pltpu.async_copy(ed_hbm.at[di], edb.at[b], sg[b])
        pltpu.async_copy(m_hbm.at[di], mb.at[b], sg[b])

        @pl.when(c == 0)
        def _():
            pltpu.async_copy(tab0.at[si], rowsb.at[b], sg[b])

        @pl.when(c == 1)
        def _():
            pltpu.async_copy(tab1.at[si], rowsb.at[b], sg[b])

    def _drain_gathers(ci, b, tab0):
        si = src2.at[ci]
        di = dst2.at[ci]
        pltpu.make_async_copy(es_hbm.at[si], esb.at[b], sg[b]).wait()
        pltpu.make_async_copy(ed_hbm.at[di], edb.at[b], sg[b]).wait()
        pltpu.make_async_copy(m_hbm.at[di], mb.at[b], sg[b]).wait()
        pltpu.make_async_copy(tab0.at[si], rowsb.at[b], sg[b]).wait()

    for p in range(n_pass):
        tab0 = xps[p]
        tab1 = xps[n_pass + p]
        do_den = (p == 0)

        def _drain_scatters(b, ci_old):
            di = dst2.at[ci_old]
            if do_den:
                pltpu.make_async_copy(exb.at[b], den_sp.at[di],
                                      ss[b]).wait()
            pltpu.make_async_copy(rowsb.at[b], acc_sp.at[di], ss[b]).wait()

        _zero_acc(do_den)
        plsc.subcore_barrier()

        for b in range(LOOK):
            _issue_gathers(b, b, tab0, tab1)

        def _iter(it, _):
            for b in range(NBUF):
                ci = it * NBUF + b
                bg = (b + LOOK) % NBUF
                cg = ci + LOOK

                @pl.when(cg >= NBUF)
                def _():
                    _drain_scatters(bg, cg - NBUF)

                @pl.when(cg < NCHUNK)
                def _():
                    _issue_gathers(cg, bg, tab0, tab1)

                _drain_gathers(ci, b, tab0)

                def _grp(g, _g):
                    sl = pl.ds(g * 16, 16)
                    t = esb[b, sl] + edb[b, sl]
                    ex = jnp.exp(_lrelu(t) - mb[b, sl])
                    if do_den:
                        exb[b, sl] = ex
                    for j in range(16):
                        w = ex[j]
                        r = g * 16 + j
                        for f in range(fb):
                            fl = pl.ds(f * 16, 16)
                            rowsb[b, r, fl] = rowsb[b, r, fl] * w
                    return 0

                lax.fori_loop(0, KCH // 16, _grp, 0)

                di = dst2.at[ci]
                if do_den:
                    pltpu.async_copy(exb.at[b], den_sp.at[di], ss[b],
                                     add=True)
                pltpu.async_copy(rowsb.at[b], acc_sp.at[di], ss[b], add=True)

            return 0

        lax.fori_loop(0, ITERS, _iter, 0)

        for cc in range(NCHUNK - (NBUF - LOOK), NCHUNK):
            _drain_scatters(cc % NBUF, cc)
        plsc.subcore_barrier()

        q = c * n_pass + p
        pltpu.sync_copy(acc_sp.at[pl.ds(base_row, rows_per_tile)],
                        numer_hbm.at[q, pl.ds(base_row, rows_per_tile)])
        if do_den:
            pltpu.sync_copy(den_sp.at[pl.ds(base_row, rows_per_tile)],
                            den_hbm.at[c, pl.ds(base_row, rows_per_tile)])


def _sc_edge(src, dst, es, ed, m, xps):
    nq = len(xps)
    n_pass = nq // 2
    f_scq = xps[0].shape[1]
    mesh = plsc.VectorSubcoreMesh(core_axis_name="c", subcore_axis_name="s",
                                  num_cores=NCORE, num_subcores=NSUB)
    kern = pl.kernel(
        functools.partial(_sc_edge_body, f_scq=f_scq, n_pass=n_pass),
        out_type=[
            jax.ShapeDtypeStruct((nq, NPAD, f_scq), jnp.float32),
            jax.ShapeDtypeStruct((NCORE, NPAD), jnp.float32),
        ],
        mesh=mesh,
        compiler_params=pltpu.CompilerParams(use_tc_tiling_on_sc=False),
        scratch_types=[
            pltpu.VMEM((NCHUNK, KCH), jnp.int32),
            pltpu.VMEM((NCHUNK, KCH), jnp.int32),
            pltpu.VMEM((NBUF, KCH), jnp.float32),
            pltpu.VMEM((NBUF, KCH), jnp.float32),
            pltpu.VMEM((NBUF, KCH), jnp.float32),
            pltpu.VMEM((NBUF, KCH), jnp.float32),
            pltpu.VMEM((NBUF, KCH, f_scq), jnp.float32),
            pltpu.SemaphoreType.DMA,
            pltpu.SemaphoreType.DMA,
            pltpu.SemaphoreType.DMA,
            pltpu.SemaphoreType.DMA,
            pltpu.SemaphoreType.DMA,
            pltpu.SemaphoreType.DMA,
            pltpu.SemaphoreType.DMA,
            pltpu.SemaphoreType.DMA,
            pltpu.SemaphoreType.DMA,
            pltpu.SemaphoreType.DMA,
            pltpu.VMEM_SHARED((NPAD, f_scq), jnp.float32),
            pltpu.VMEM_SHARED((NPAD,), jnp.float32),
        ],
    )
    src_p = jnp.pad(src, (0, EPAD - E_TOTAL), constant_values=PAD_NODE)
    dst_p = jnp.pad(dst, (0, EPAD - E_TOTAL), constant_values=PAD_NODE)
    return kern(src_p.reshape(NSUB, NCHUNK, KCH),
                dst_p.reshape(NSUB, NCHUNK, KCH),
                es, ed, m, *xps)


def _split(v, nq, axis=0):
    step = v.shape[axis] // nq
    return [lax.slice_in_dim(v, q * step, (q + 1) * step, axis=axis)
            for q in range(nq)]


def kernel(x, edge_index, batch, W1, b1, W2, b2, W3, b3,
           Wg1, as1, ad1, bg1, Wg2, as2, ad2, bg2, Wg3, as3, ad3, bg3,
           Wg4, as4, ad4, bg4, Wg5, as5, ad5, bg5, Wl, bl):
    pad = NPAD - N_NODES
    xp_ = jnp.pad(x, ((0, pad), (0, 0)))
    batch_p = jnp.pad(batch, (0, pad), constant_values=G).astype(jnp.int32)
    src = edge_index[0]
    dst = edge_index[1]

    row = lambda v: v.reshape(1, -1)
    flat = lambda v: v.reshape(-1)

    outs = _tc_prep(xp_, W1, row(b1), W2, row(b2), W3, row(b3),
                    Wg1, row(as1), row(ad1))
    xps, (es, ed, m) = outs[:-3], outs[-3:]

    layer_next = [(Wg2, as2, ad2, bg1), (Wg3, as3, ad3, bg2),
                  (Wg4, as4, ad4, bg3), (Wg5, as5, ad5, bg4)]

    for wnext, asn, adn, b_prev in layer_next:
        numer, den = _sc_edge(src, dst, flat(es), flat(ed), flat(m), xps)
        nq_in = numer.shape[0]
        n_parts = [numer[q] for q in range(nq_in)]
        outs = _tc_mid(n_parts, den[0].reshape(NPAD, 1),
                       [row(p) for p in _split(b_prev, nq_in)],
                       _split(wnext, nq_in), row(asn), row(adn))
        xps, (es, ed, m) = outs[:-3], outs[-3:]

    numer, den = _sc_edge(src, dst, flat(es), flat(ed), flat(m), xps)
    nq_in = numer.shape[0]
    out = _tc_final([numer[q] for q in range(nq_in)],
                    den[0].reshape(NPAD, 1),
                    [row(p) for p in _split(bg5, nq_in)],
                    batch_p.reshape(NPAD, 1), Wl, row(bl))
    return out


# speedup vs baseline: 1.2464x; 1.2464x over previous
import functools

import jax
import jax.numpy as jnp
from jax import lax
from jax.experimental import pallas as pl
from jax.experimental.pallas import tpu as pltpu
from jax.experimental.pallas import tpu_sc as plsc

N_NODES = 10000
NPAD = 10240
RB = 512
NBLK = NPAD // RB
E_TOTAL = 640000
NSUB = 16
NCORE = 2
KCH = 80
G = 64
EPS = 1e-16
LRELU_SLOPE = 0.2

NBUF = 5
LOOK = 3
NCHUNK = 500
EPAD = NSUB * NCHUNK * KCH
PAD_NODE = N_NODES + 100
ITERS = NCHUNK // NBUF


def _nq(f_out):
    if f_out >= 256:
        return 8
    return 4 if f_out >= 64 else 2


def _lrelu(t):
    return jnp.where(t > 0, t, LRELU_SLOPE * t)


def _attn_tail(xp, a_s, a_d, i, xp_refs, es_ref, ed_ref, g_ref, gmax_sm):
    nq = len(xp_refs)
    fq = xp.shape[1] // nq

    @pl.when(i < NBLK)
    def _():
        es = jnp.sum(xp * a_s, axis=1, keepdims=True)
        ed = jnp.sum(xp * a_d, axis=1, keepdims=True)
        for q, r in enumerate(xp_refs):
            r[...] = xp[:, q * fq:(q + 1) * fq]
        es_ref[...] = es
        ed_ref[...] = ed
        cur = jnp.max(es)
        prev = jnp.where(i == 0, -jnp.inf, gmax_sm[0])
        gmax_sm[0] = jnp.maximum(prev, cur)

    @pl.when(i == NBLK)
    def _():
        g_ref[...] = jnp.full((8, 128), gmax_sm[0], jnp.float32)


def _prep_body(*refs, nq_out):
    (x_ref, w1, b1, w2, b2, w3, b3, wg, a_s, a_d) = refs[:10]
    xp_refs = refs[10:10 + nq_out]
    es_ref, ed_ref, g_ref, gmax_sm = refs[10 + nq_out:]
    i = pl.program_id(0)
    h = jax.nn.relu(jnp.dot(x_ref[...], w1[...],
                            preferred_element_type=jnp.float32) + b1[...])
    h = jax.nn.relu(jnp.dot(h, w2[...], preferred_element_type=jnp.float32)
                    + b2[...])
    h = jax.nn.relu(jnp.dot(h, w3[...], preferred_element_type=jnp.float32)
                    + b3[...])
    xp = jnp.dot(h, wg[...], preferred_element_type=jnp.float32)
    _attn_tail(xp, a_s[...], a_d[...], i, xp_refs, es_ref, ed_ref, g_ref,
               gmax_sm)


def _mid_body(*refs, nq_in, nq_out):
    n_refs = refs[:nq_in]
    den_ref = refs[nq_in]
    b_refs = refs[nq_in + 1:2 * nq_in + 1]
    w_refs = refs[2 * nq_in + 1:3 * nq_in + 1]
    a_s, a_d = refs[3 * nq_in + 1:3 * nq_in + 3]
    xp_refs = refs[3 * nq_in + 3:3 * nq_in + 3 + nq_out]
    es_ref, ed_ref, g_ref, gmax_sm = refs[3 * nq_in + 3 + nq_out:]
    i = pl.program_id(0)
    d = den_ref[...] + EPS
    xp = None
    for q in range(nq_in):
        h = jax.nn.relu(n_refs[q][...] / d + b_refs[q][...])
        part = jnp.dot(h, w_refs[q][...], preferred_element_type=jnp.float32)
        xp = part if xp is None else xp + part
    _attn_tail(xp, a_s[...], a_d[...], i, xp_refs, es_ref, ed_ref, g_ref,
               gmax_sm)


def _final_body(*refs, nq_in):
    n_refs = refs[:nq_in]
    den_ref = refs[nq_in]
    b_refs = refs[nq_in + 1:2 * nq_in + 1]
    batch_ref, wl, bl, out_ref, sums_sc, cnt_sc = refs[2 * nq_in + 1:]
    i = pl.program_id(0)
    fq = n_refs[0].shape[1]

    @pl.when(i == 0)
    def _():
        sums_sc[...] = jnp.zeros_like(sums_sc)
        cnt_sc[...] = jnp.zeros_like(cnt_sc)

    @pl.when(i < NBLK)
    def _():
        d = den_ref[...] + EPS
        iota = lax.broadcasted_iota(jnp.int32, (RB, G), 1)
        oh = (batch_ref[...] == iota).astype(jnp.float32)
        dn = (((0,), (0,)), ((), ()))
        for q in range(nq_in):
            h = jax.nn.relu(n_refs[q][...] / d + b_refs[q][...])
            sums_sc[:, q * fq:(q + 1) * fq] += lax.dot_general(
                oh, h, dn, preferred_element_type=jnp.float32)
        ones = jnp.ones((RB, 1), jnp.float32)
        cnt_sc[...] += lax.dot_general(oh, ones, dn,
                                       preferred_element_type=jnp.float32)

    @pl.when(i == NBLK)
    def _():
        pooled = sums_sc[...] / jnp.maximum(cnt_sc[...], 1.0)
        z = jnp.dot(pooled, wl[...], preferred_element_type=jnp.float32) \
            + bl[...]
        out_ref[...] = jax.nn.sigmoid(z)


def _clamp_map(i):
    return (jnp.minimum(i, NBLK - 1), 0)


def _full(shape):
    return pl.BlockSpec(shape, lambda i: tuple(0 for _ in shape))


def _node_specs(f_out):
    nq = _nq(f_out)
    fq = f_out // nq
    out_shape = [jax.ShapeDtypeStruct((NPAD, fq), jnp.float32)
                 for _ in range(nq)] + [
        jax.ShapeDtypeStruct((NPAD, 1), jnp.float32),
        jax.ShapeDtypeStruct((NPAD, 1), jnp.float32),
        jax.ShapeDtypeStruct((8, 128), jnp.float32),
    ]
    out_specs = [pl.BlockSpec((RB, fq), _clamp_map) for _ in range(nq)] + [
        pl.BlockSpec((RB, 1), _clamp_map),
        pl.BlockSpec((RB, 1), _clamp_map),
        pl.BlockSpec((8, 128), lambda i: (0, 0)),
    ]
    scratch = [pltpu.SMEM((1,), jnp.float32)]
    return out_shape, out_specs, scratch


def _tc_prep(x, w1, b1, w2, b2, w3, b3, wg, a_s, a_d):
    f_out = wg.shape[1]
    nq = _nq(f_out)
    out_shape, out_specs, scratch = _node_specs(f_out)
    args = (x, w1, b1, w2, b2, w3, b3, wg, a_s, a_d)
    in_specs = [pl.BlockSpec((RB, 1), _clamp_map)] + [
        _full(a.shape) for a in args[1:]]
    return pl.pallas_call(
        functools.partial(_prep_body, nq_out=nq),
        grid=(NBLK + 1,),
        in_specs=in_specs,
        out_specs=out_specs,
        out_shape=out_shape,
        scratch_shapes=scratch,
    )(*args)


def _tc_mid(n_parts, den, b_parts, w_parts, a_s, a_d):
    nq_in = len(n_parts)
    f_out = w_parts[0].shape[1]
    nq_out = _nq(f_out)
    fq_in = n_parts[0].shape[1]
    out_shape, out_specs, scratch = _node_specs(f_out)
    in_specs = ([pl.BlockSpec((RB, fq_in), _clamp_map)] * nq_in
                + [pl.BlockSpec((RB, 1), _clamp_map)]
                + [_full(a.shape) for a in (*b_parts, *w_parts, a_s, a_d)])
    return pl.pallas_call(
        functools.partial(_mid_body, nq_in=nq_in, nq_out=nq_out),
        grid=(NBLK + 1,),
        in_specs=in_specs,
        out_specs=out_specs,
        out_shape=out_shape,
        scratch_shapes=scratch,
    )(*n_parts, den, *b_parts, *w_parts, a_s, a_d)


def _tc_final(n_parts, den, b_parts, batch, wl, bl):
    nq_in = len(n_parts)
    fq_in = n_parts[0].shape[1]
    in_specs = ([pl.BlockSpec((RB, fq_in), _clamp_map)] * nq_in
                + [pl.BlockSpec((RB, 1), _clamp_map)]
                + [_full(a.shape) for a in b_parts]
                + [pl.BlockSpec((RB, 1), _clamp_map),
                   _full(wl.shape), _full(bl.shape)])
    return pl.pallas_call(
        functools.partial(_final_body, nq_in=nq_in),
        grid=(NBLK + 1,),
        in_specs=in_specs,
        out_specs=pl.BlockSpec((G, 1), lambda i: (0, 0)),
        out_shape=jax.ShapeDtypeStruct((G, 1), jnp.float32),
        scratch_shapes=[pltpu.VMEM((G, nq_in * fq_in), jnp.float32),
                        pltpu.VMEM((G, 1), jnp.float32)],
    )(*n_parts, den, *b_parts, batch, wl, bl)


def _sc_edge_body(*refs, f_scq, n_pass):
    nq = 2 * n_pass
    (src_hbm, dst_hbm, es_hbm, ed_hbm, g_hbm) = refs[:5]
    xps = refs[5:5 + nq]
    numer_hbm, den_hbm = refs[5 + nq:7 + nq]
    (src2, dst2, esb, edb, gvb, exb, rowsb,
     sg0, sg1, sg2, sg3, sg4, ss0, ss1, ss2, ss3, ss4,
     acc_sp, den_sp) = refs[7 + nq:]
    c = lax.axis_index("c")
    s = lax.axis_index("s")
    sg = (sg0, sg1, sg2, sg3, sg4)
    ss = (ss0, ss1, ss2, ss3, ss4)
    rows_per_tile = NPAD // NSUB
    base_row = s * rows_per_tile
    fb = f_scq // 16

    pltpu.sync_copy(src_hbm.at[s], src2)
    pltpu.sync_copy(dst_hbm.at[s], dst2)
    pltpu.sync_copy(g_hbm.at[pl.ds(0, 16)], gvb)

    def _zero_acc(zero_den):
        def _zero_rows(j, _):
            for f in range(fb):
                rowsb[0, j, pl.ds(f * 16, 16)] = jnp.zeros((16,), jnp.float32)
            return 0

        lax.fori_loop(0, KCH, _zero_rows, 0)
        offs = [z * KCH for z in range(rows_per_tile // KCH)]
        if (rows_per_tile // KCH) * KCH < rows_per_tile:
            offs.append(rows_per_tile - KCH)
        for o in offs:
            pltpu.sync_copy(rowsb.at[0],
                            acc_sp.at[pl.ds(base_row + o, KCH)])
        if zero_den:
            for z in range(KCH // 16):
                exb[0, pl.ds(z * 16, 16)] = jnp.zeros((16,), jnp.float32)
            for o in offs:
                pltpu.sync_copy(exb.at[0],
                                den_sp.at[pl.ds(base_row + o, KCH)])

    def _issue_gathers(ci, b, tab0, tab1):
        si = src2.at[ci]
        di = dst2.at[ci]
        pltpu.async_copy(es_hbm.at[si], esb.at[b], sg[b])
        pltpu.async_copy(ed_hbm.at[di], edb.at[b], sg[b])

        @pl.when(c == 0)
        def _():
            pltpu.async_copy(tab0.at[si], rowsb.at[b], sg[b])

        @pl.when(c == 1)
        def _():
            pltpu.async_copy(tab1.at[si], rowsb.at[b], sg[b])

    def _drain_gathers(ci, b, tab0):
        si = src2.at[ci]
        di = dst2.at[ci]
        pltpu.make_async_copy(es_hbm.at[si], esb.at[b], sg[b]).wait()
        pltpu.make_async_copy(ed_hbm.at[di], edb.at[b], sg[b]).wait()
        pltpu.make_async_copy(tab0.at[si], rowsb.at[b], sg[b]).wait()

    for p in range(n_pass):
        tab0 = xps[p]
        tab1 = xps[n_pass + p]
        do_den = (p == 0)

        def _drain_scatters(b, ci_old):
            di = dst2.at[ci_old]
            if do_den:
                pltpu.make_async_copy(exb.at[b], den_sp.at[di],
                                      ss[b]).wait()
            pltpu.make_async_copy(rowsb.at[b], acc_sp.at[di], ss[b]).wait()

        _zero_acc(do_den)
        plsc.subcore_barrier()

        for b in range(LOOK):
            _issue_gathers(b, b, tab0, tab1)

        def _iter(it, _):
            for b in range(NBUF):
                ci = it * NBUF + b
                bg = (b + LOOK) % NBUF
                cg = ci + LOOK

                @pl.when(cg >= NBUF)
                def _():
                    _drain_scatters(bg, cg - NBUF)

                @pl.when(cg < NCHUNK)
                def _():
                    _issue_gathers(cg, bg, tab0, tab1)

                _drain_gathers(ci, b, tab0)

                def _grp(g, _g):
                    sl = pl.ds(g * 16, 16)
                    ed_v = edb[b, sl]
                    t = esb[b, sl] + ed_v
                    ex = jnp.exp(_lrelu(t) - _lrelu(gvb[...] + ed_v))
                    if do_den:
                        exb[b, sl] = ex
                    for j in range(16):
                        w = ex[j]
                        r = g * 16 + j
                        for f in range(fb):
                            fl = pl.ds(f * 16, 16)
                            rowsb[b, r, fl] = rowsb[b, r, fl] * w
                    return 0

                lax.fori_loop(0, KCH // 16, _grp, 0)

                di = dst2.at[ci]
                if do_den:
                    pltpu.async_copy(exb.at[b], den_sp.at[di], ss[b],
                                     add=True)
                pltpu.async_copy(rowsb.at[b], acc_sp.at[di], ss[b], add=True)

            return 0

        lax.fori_loop(0, ITERS, _iter, 0)

        for cc in range(NCHUNK - (NBUF - LOOK), NCHUNK):
            _drain_scatters(cc % NBUF, cc)
        plsc.subcore_barrier()

        q = c * n_pass + p
        pltpu.sync_copy(acc_sp.at[pl.ds(base_row, rows_per_tile)],
                        numer_hbm.at[q, pl.ds(base_row, rows_per_tile)])
        if do_den:
            pltpu.sync_copy(den_sp.at[pl.ds(base_row, rows_per_tile)],
                            den_hbm.at[c, pl.ds(base_row, rows_per_tile)])


def _sc_edge(src, dst, es, ed, g, xps):
    nq = len(xps)
    n_pass = nq // 2
    f_scq = xps[0].shape[1]
    mesh = plsc.VectorSubcoreMesh(core_axis_name="c", subcore_axis_name="s",
                                  num_cores=NCORE, num_subcores=NSUB)
    kern = pl.kernel(
        functools.partial(_sc_edge_body, f_scq=f_scq, n_pass=n_pass),
        out_type=[
            jax.ShapeDtypeStruct((nq, NPAD, f_scq), jnp.float32),
            jax.ShapeDtypeStruct((NCORE, NPAD), jnp.float32),
        ],
        mesh=mesh,
        compiler_params=pltpu.CompilerParams(use_tc_tiling_on_sc=False),
        scratch_types=[
            pltpu.VMEM((NCHUNK, KCH), jnp.int32),
            pltpu.VMEM((NCHUNK, KCH), jnp.int32),
            pltpu.VMEM((NBUF, KCH), jnp.float32),
            pltpu.VMEM((NBUF, KCH), jnp.float32),
            pltpu.VMEM((16,), jnp.float32),
            pltpu.VMEM((NBUF, KCH), jnp.float32),
            pltpu.VMEM((NBUF, KCH, f_scq), jnp.float32),
            pltpu.SemaphoreType.DMA,
            pltpu.SemaphoreType.DMA,
            pltpu.SemaphoreType.DMA,
            pltpu.SemaphoreType.DMA,
            pltpu.SemaphoreType.DMA,
            pltpu.SemaphoreType.DMA,
            pltpu.SemaphoreType.DMA,
            pltpu.SemaphoreType.DMA,
            pltpu.SemaphoreType.DMA,
            pltpu.SemaphoreType.DMA,
            pltpu.VMEM_SHARED((NPAD, f_scq), jnp.float32),
            pltpu.VMEM_SHARED((NPAD,), jnp.float32),
        ],
    )
    src_p = jnp.pad(src, (0, EPAD - E_TOTAL), constant_values=PAD_NODE)
    dst_p = jnp.pad(dst, (0, EPAD - E_TOTAL), constant_values=PAD_NODE)
    return kern(src_p.reshape(NSUB, NCHUNK, KCH),
                dst_p.reshape(NSUB, NCHUNK, KCH),
                es, ed, g, *xps)


def _split(v, nq, axis=0):
    step = v.shape[axis] // nq
    return [lax.slice_in_dim(v, q * step, (q + 1) * step, axis=axis)
            for q in range(nq)]


def kernel(x, edge_index, batch, W1, b1, W2, b2, W3, b3,
           Wg1, as1, ad1, bg1, Wg2, as2, ad2, bg2, Wg3, as3, ad3, bg3,
           Wg4, as4, ad4, bg4, Wg5, as5, ad5, bg5, Wl, bl):
    pad = NPAD - N_NODES
    xp_ = jnp.pad(x, ((0, pad), (0, 0)))
    batch_p = jnp.pad(batch, (0, pad), constant_values=G).astype(jnp.int32)
    src = edge_index[0]
    dst = edge_index[1]

    row = lambda v: v.reshape(1, -1)
    flat = lambda v: v.reshape(-1)

    outs = _tc_prep(xp_, W1, row(b1), W2, row(b2), W3, row(b3),
                    Wg1, row(as1), row(ad1))
    xps, (es, ed, m) = outs[:-3], outs[-3:]

    layer_next = [(Wg2, as2, ad2, bg1), (Wg3, as3, ad3, bg2),
                  (Wg4, as4, ad4, bg3), (Wg5, as5, ad5, bg4)]

    for wnext, asn, adn, b_prev in layer_next:
        numer, den = _sc_edge(src, dst, flat(es), flat(ed), flat(m), xps)
        nq_in = numer.shape[0]
        n_parts = [numer[q] for q in range(nq_in)]
        outs = _tc_mid(n_parts, den[0].reshape(NPAD, 1),
                       [row(p) for p in _split(b_prev, nq_in)],
                       _split(wnext, nq_in), row(asn), row(adn))
        xps, (es, ed, m) = outs[:-3], outs[-3:]

    numer, den = _sc_edge(src, dst, flat(es), flat(ed), flat(m), xps)
    nq_in = numer.shape[0]
    out = _tc_final([numer[q] for q in range(nq_in)],
                    den[0].reshape(NPAD, 1),
                    [row(p) for p in _split(bg5, nq_in)],
                    batch_p.reshape(NPAD, 1), Wl, row(bl))
    return out
